# baseline (speedup 1.0000x reference)
import functools

import jax
import jax.numpy as jnp
from jax import lax
from jax.experimental import pallas as pl
from jax.experimental.pallas import tpu as pltpu
from jax.experimental.pallas import tpu_sc as plsc

_VOCAB = 100000
_D = 128
_MAX_HOP = 3
_B = 128
_MEM = 200
_TOK = 4
_CONV_MAX = 100

_NT = _MAX_HOP + 1

_NC = 2
_NS = 16
_NW = _NC * _NS

_NSLICE = 4
_BS = _B // _NSLICE
_N_OUT = _NT * _BS * _MEM
_N_IN = _N_OUT * _TOK

_CH_IN = 128
_CH_OUT = _CH_IN // _TOK
_OUT_PER_W = _N_OUT // _NW
_N_CHUNKS = _N_IN // _NW // _CH_IN
_N_PAIRS = _N_CHUNKS // 2
_HAS_TAIL = _N_CHUNKS % 2

_BB = 16


_N_BUF = 4


def _sc_body(idx_hbm, table_hbm, out_hbm,
             idx_v, r0, r1, r2, r3, out_a, out_b,
             g0, g1, g2, g3, sem_wa, sem_wb):
    c = lax.axis_index("c")
    s = lax.axis_index("s")
    wid = s * _NC + c
    out_base = wid * _OUT_PER_W

    rows = [r0, r1, r2, r3]
    gsems = [g0, g1, g2, g3]
    outs = [out_a, out_b]
    wsems = [sem_wa, sem_wb]

    pltpu.sync_copy(idx_hbm.at[wid], idx_v)

    def fire_gather(buf, sem, g):
        pltpu.async_copy(table_hbm.at[idx_v.at[g]], buf, sem)

    def drain_gather(buf, sem):
        pltpu.make_async_copy(table_hbm.at[pl.ds(0, _CH_IN)], buf, sem).wait()

    def drain_write(buf, sem):
        ob = pl.multiple_of(out_base, 8)
        pltpu.make_async_copy(buf, out_hbm.at[pl.ds(ob, _CH_OUT)],
                              sem).wait()

    def sum_chunk(buf, out):
        def row_body(r, carry):
            for cc in range(_D // 16):
                sl = pl.ds(cc * 16, 16)
                out[r, sl] = (buf[4 * r, sl] + buf[4 * r + 1, sl]
                              + buf[4 * r + 2, sl] + buf[4 * r + 3, sl])
            return carry
        lax.fori_loop(0, _CH_OUT, row_body, 0, unroll=4)

    for j in range(_N_BUF):
        fire_gather(rows[j], gsems[j], j)
    for j in range(_N_CHUNKS):
        b = j % _N_BUF
        p = j % 2
        drain_gather(rows[b], gsems[b])
        if j >= 2:
            drain_write(outs[p], wsems[p])
        sum_chunk(rows[b], outs[p])
        pltpu.async_copy(
            outs[p],
            out_hbm.at[pl.ds(pl.multiple_of(out_base + j * _CH_OUT, 8),
                             _CH_OUT)],
            wsems[p])
        if j + _N_BUF < _N_CHUNKS:
            fire_gather(rows[b], gsems[b], j + _N_BUF)
    drain_write(outs[0], wsems[0])
    drain_write(outs[1], wsems[1])


@functools.lru_cache(maxsize=1)
def _sc_gather_pool():
    return pl.kernel(
        _sc_body,
        mesh=plsc.VectorSubcoreMesh(core_axis_name="c", subcore_axis_name="s",
                                    num_cores=_NC, num_subcores=_NS),
        out_type=jax.ShapeDtypeStruct((_N_OUT, _D), jnp.float32),
        scratch_types=[
            pltpu.VMEM((_N_CHUNKS, _CH_IN), jnp.int32),
            pltpu.VMEM((_CH_IN, _D), jnp.float32),
            pltpu.VMEM((_CH_IN, _D), jnp.float32),
            pltpu.VMEM((_CH_IN, _D), jnp.float32),
            pltpu.VMEM((_CH_IN, _D), jnp.float32),
            pltpu.VMEM((_CH_OUT, _D), jnp.float32),
            pltpu.VMEM((_CH_OUT, _D), jnp.float32),
            pltpu.SemaphoreType.DMA,
            pltpu.SemaphoreType.DMA,
            pltpu.SemaphoreType.DMA,
            pltpu.SemaphoreType.DMA,
            pltpu.SemaphoreType.DMA,
            pltpu.SemaphoreType.DMA,
        ],
    )


def _tc_body(e0_ref, e1_ref, e2_ref, e3_ref, dh_ref, kb_ref, conv_ref,
             u0_ref, q_ref, ps_ref, plg_ref, gp_ref, kr_ref):
    dh = dh_ref[...]
    kb3 = kb_ref[...][:, :, :1]
    conv3 = conv_ref[...][:, :, :1]

    m_iota3 = lax.broadcasted_iota(jnp.int32, (_BB, _MEM, _CONV_MAX), 1)
    j_iota3 = lax.broadcasted_iota(jnp.int32, (_BB, _MEM, _CONV_MAX), 2)
    hid3 = m_iota3 - kb3
    onehot = ((j_iota3 == hid3) & (hid3 >= 0)
              & (hid3 < conv3)).astype(jnp.float32)
    lm = lax.dot_general(onehot, dh, (((2,), (1,)), ((0,), (0,))),
                         precision=lax.Precision.HIGHEST)

    lm_flat = lm.reshape(_BB * _MEM, _D)
    Eb = [e0_ref[...] + lm_flat, e1_ref[...] + lm_flat,
          e2_ref[...] + lm_flat, e3_ref[...] + lm_flat]
    Eb = [e.reshape(_BB, _MEM, _D) for e in Eb]

    def attend(e, u):
        return lax.dot_general(e, u, (((2,), (1,)), ((0,), (0,))),
                               precision=lax.Precision.HIGHEST)

    def pool(e, p):
        return lax.dot_general(p, e, (((1,), (1,)), ((0,), (0,))),
                               precision=lax.Precision.HIGHEST)

    u = u0_ref[...]
    logit = None
    for hop in range(_MAX_HOP):
        logit = attend(Eb[hop], u)
        p = jax.nn.softmax(logit, axis=1)
        u = u + pool(Eb[hop + 1], p)
    gp = jax.nn.sigmoid(logit)
    gp_ref[...] = gp
    kr_ref[...] = u

    u2 = q_ref[...]
    p2 = None
    logits2 = None
    for hop in range(_MAX_HOP):
        logits2 = gp * attend(Eb[hop], u2)
        p2 = jax.nn.softmax(logits2, axis=1)
        u2 = u2 + pool(Eb[hop + 1], p2 * gp)
    ps_ref[...] = p2
    plg_ref[...] = logits2


def _e_spec(t):
    return pl.BlockSpec((_BB * _MEM, _D),
                        lambda i, t=t: (t * (_BS // _BB) + i, 0))


def _tc_attention(e, dh_outputs, kb2, conv2, u0, q):
    grid = (_BS // _BB,)
    return pl.pallas_call(
        _tc_body,
        grid=grid,
        in_specs=[
            _e_spec(0), _e_spec(1), _e_spec(2), _e_spec(3),
            pl.BlockSpec((_BB, _CONV_MAX, _D), lambda i: (i, 0, 0)),
            pl.BlockSpec((_BB, 1, _D), lambda i: (i, 0, 0)),
            pl.BlockSpec((_BB, 1, _D), lambda i: (i, 0, 0)),
            pl.BlockSpec((_BB, _D), lambda i: (i, 0)),
            pl.BlockSpec((_BB, _D), lambda i: (i, 0)),
        ],
        out_specs=[
            pl.BlockSpec((_BB, _MEM), lambda i: (i, 0)),
            pl.BlockSpec((_BB, _MEM), lambda i: (i, 0)),
            pl.BlockSpec((_BB, _MEM), lambda i: (i, 0)),
            pl.BlockSpec((_BB, _D), lambda i: (i, 0)),
        ],
        out_shape=[
            jax.ShapeDtypeStruct((_BS, _MEM), jnp.float32),
            jax.ShapeDtypeStruct((_BS, _MEM), jnp.float32),
            jax.ShapeDtypeStruct((_BS, _MEM), jnp.float32),
            jax.ShapeDtypeStruct((_BS, _D), jnp.float32),
        ],
    )(e, e, e, e, dh_outputs, kb2, conv2, u0, q)


@jax.jit
def kernel(story, kb_len, conv_len, dh_hidden, dh_outputs, query_vector, C):
    table = C.reshape(_NT * _VOCAB, _D)
    kb2 = jnp.broadcast_to(kb_len[:, None, None], (_B, 1, _D)).astype(jnp.int32)
    conv2 = jnp.broadcast_to(conv_len[:, None, None],
                             (_B, 1, _D)).astype(jnp.int32)
    u0 = dh_hidden[0]
    toff = (jnp.arange(_NT, dtype=jnp.int32) * _VOCAB)[:, None]

    outs = []
    for sl in range(_NSLICE):
        b0 = sl * _BS
        flat = story[b0:b0 + _BS].reshape(-1).astype(jnp.int32)
        idx = toff + flat[None, :]
        idx = idx.reshape(_NW, _N_CHUNKS, _CH_IN)
        e = _sc_gather_pool()(idx, table)
        outs.append(_tc_attention(
            e, dh_outputs[b0:b0 + _BS], kb2[b0:b0 + _BS], conv2[b0:b0 + _BS],
            u0[b0:b0 + _BS], query_vector[b0:b0 + _BS]))

    prob_soft, prob_logits, global_pointer, kb_readout = (
        jnp.concatenate([o[k] for o in outs], axis=0) for k in range(4))
    return prob_soft, prob_logits, global_pointer, kb_readout


# speedup vs baseline: 1.1172x; 1.1172x over previous
import functools

import jax
import jax.numpy as jnp
from jax import lax
from jax.experimental import pallas as pl
from jax.experimental.pallas import tpu as pltpu
from jax.experimental.pallas import tpu_sc as plsc

_VOCAB = 100000
_D = 128
_MAX_HOP = 3
_B = 128
_MEM = 200
_TOK = 4
_CONV_MAX = 100

_NT = _MAX_HOP + 1

_NC = 2
_NS = 16
_NW = _NC * _NS

_NSLICE = 4
_BS = _B // _NSLICE
_N_OUT = _NT * _BS * _MEM
_N_IN = _N_OUT * _TOK

_CH_IN = 128
_CH_OUT = _CH_IN // _TOK
_OUT_PER_W = _N_OUT // _NW
_N_CHUNKS = _N_IN // _NW // _CH_IN
_N_PAIRS = _N_CHUNKS // 2
_HAS_TAIL = _N_CHUNKS % 2

_BB = 16


_N_BUF = 4


def _sc_body(idx_hbm, table_hbm, out_hbm,
             idx_v, r0, r1, r2, r3, out_a, out_b,
             g0, g1, g2, g3, sem_wa, sem_wb):
    c = lax.axis_index("c")
    s = lax.axis_index("s")
    wid = s * _NC + c
    out_base = wid * _OUT_PER_W

    rows = [r0, r1, r2, r3]
    gsems = [g0, g1, g2, g3]
    outs = [out_a, out_b]
    wsems = [sem_wa, sem_wb]

    pltpu.sync_copy(idx_hbm.at[wid], idx_v)

    def fire_gather(buf, sem, g):
        pltpu.async_copy(table_hbm.at[idx_v.at[g]], buf, sem)

    def drain_gather(buf, sem):
        pltpu.make_async_copy(table_hbm.at[pl.ds(0, _CH_IN)], buf, sem).wait()

    def drain_write(buf, sem):
        ob = pl.multiple_of(out_base, 8)
        pltpu.make_async_copy(buf, out_hbm.at[pl.ds(ob, _CH_OUT)],
                              sem).wait()

    def sum_chunk(buf, out):
        def row_body(r, carry):
            for cc in range(_D // 16):
                sl = pl.ds(cc * 16, 16)
                out[r, sl] = (buf[4 * r, sl] + buf[4 * r + 1, sl]
                              + buf[4 * r + 2, sl] + buf[4 * r + 3, sl])
            return carry
        lax.fori_loop(0, _CH_OUT, row_body, 0, unroll=4)

    for j in range(_N_BUF):
        fire_gather(rows[j], gsems[j], j)

    def group_body(i, carry):
        for k in range(_N_BUF):
            j = _N_BUF * i + k
            drain_gather(rows[k], gsems[k])
            if k >= 2:
                drain_write(outs[k % 2], wsems[k % 2])
            else:
                @pl.when(i > 0)
                def _():
                    drain_write(outs[k % 2], wsems[k % 2])
            sum_chunk(rows[k], outs[k % 2])
            pltpu.async_copy(
                outs[k % 2],
                out_hbm.at[pl.ds(
                    pl.multiple_of(out_base + j * _CH_OUT, 8), _CH_OUT)],
                wsems[k % 2])

            @pl.when(j + _N_BUF < _N_CHUNKS)
            def _():
                fire_gather(rows[k], gsems[k], j + _N_BUF)
        return carry

    lax.fori_loop(0, _N_CHUNKS // _N_BUF, group_body, 0, unroll=False)
    for j in range(_N_CHUNKS - _N_CHUNKS % _N_BUF, _N_CHUNKS):
        b = j % _N_BUF
        drain_gather(rows[b], gsems[b])
        drain_write(outs[j % 2], wsems[j % 2])
        sum_chunk(rows[b], outs[j % 2])
        pltpu.async_copy(
            outs[j % 2],
            out_hbm.at[pl.ds(pl.multiple_of(out_base + j * _CH_OUT, 8),
                             _CH_OUT)],
            wsems[j % 2])
    drain_write(outs[0], wsems[0])
    drain_write(outs[1], wsems[1])


@functools.lru_cache(maxsize=1)
def _sc_gather_pool():
    return pl.kernel(
        _sc_body,
        mesh=plsc.VectorSubcoreMesh(core_axis_name="c", subcore_axis_name="s",
                                    num_cores=_NC, num_subcores=_NS),
        out_type=jax.ShapeDtypeStruct((_N_OUT, _D), jnp.float32),
        scratch_types=[
            pltpu.VMEM((_N_CHUNKS, _CH_IN), jnp.int32),
            pltpu.VMEM((_CH_IN, _D), jnp.float32),
            pltpu.VMEM((_CH_IN, _D), jnp.float32),
            pltpu.VMEM((_CH_IN, _D), jnp.float32),
            pltpu.VMEM((_CH_IN, _D), jnp.float32),
            pltpu.VMEM((_CH_OUT, _D), jnp.float32),
            pltpu.VMEM((_CH_OUT, _D), jnp.float32),
            pltpu.SemaphoreType.DMA,
            pltpu.SemaphoreType.DMA,
            pltpu.SemaphoreType.DMA,
            pltpu.SemaphoreType.DMA,
            pltpu.SemaphoreType.DMA,
            pltpu.SemaphoreType.DMA,
        ],
    )


def _tc_body(e0_ref, e1_ref, e2_ref, e3_ref, dh_ref, kb_ref, conv_ref,
             u0_ref, q_ref, ps_ref, plg_ref, gp_ref, kr_ref):
    dh = dh_ref[...]
    kb3 = kb_ref[...][:, :, :1]
    conv3 = conv_ref[...][:, :, :1]

    m_iota3 = lax.broadcasted_iota(jnp.int32, (_BB, _MEM, _CONV_MAX), 1)
    j_iota3 = lax.broadcasted_iota(jnp.int32, (_BB, _MEM, _CONV_MAX), 2)
    hid3 = m_iota3 - kb3
    onehot = ((j_iota3 == hid3) & (hid3 >= 0)
              & (hid3 < conv3)).astype(jnp.float32)
    lm = lax.dot_general(onehot, dh, (((2,), (1,)), ((0,), (0,))),
                         precision=lax.Precision.HIGHEST)

    lm_flat = lm.reshape(_BB * _MEM, _D)
    Eb = [e0_ref[...] + lm_flat, e1_ref[...] + lm_flat,
          e2_ref[...] + lm_flat, e3_ref[...] + lm_flat]
    Eb = [e.reshape(_BB, _MEM, _D) for e in Eb]

    def attend(e, u):
        return lax.dot_general(e, u, (((2,), (1,)), ((0,), (0,))),
                               precision=lax.Precision.HIGHEST)

    def pool(e, p):
        return lax.dot_general(p, e, (((1,), (1,)), ((0,), (0,))),
                               precision=lax.Precision.HIGHEST)

    u = u0_ref[...]
    logit = None
    for hop in range(_MAX_HOP):
        logit = attend(Eb[hop], u)
        p = jax.nn.softmax(logit, axis=1)
        u = u + pool(Eb[hop + 1], p)
    gp = jax.nn.sigmoid(logit)
    gp_ref[...] = gp
    kr_ref[...] = u

    u2 = q_ref[...]
    p2 = None
    logits2 = None
    for hop in range(_MAX_HOP):
        logits2 = gp * attend(Eb[hop], u2)
        p2 = jax.nn.softmax(logits2, axis=1)
        u2 = u2 + pool(Eb[hop + 1], p2 * gp)
    ps_ref[...] = p2
    plg_ref[...] = logits2


def _e_spec(t):
    return pl.BlockSpec((_BB * _MEM, _D),
                        lambda i, t=t: (t * (_BS // _BB) + i, 0))


def _tc_attention(e, dh_outputs, kb2, conv2, u0, q):
    grid = (_BS // _BB,)
    return pl.pallas_call(
        _tc_body,
        grid=grid,
        in_specs=[
            _e_spec(0), _e_spec(1), _e_spec(2), _e_spec(3),
            pl.BlockSpec((_BB, _CONV_MAX, _D), lambda i: (i, 0, 0)),
            pl.BlockSpec((_BB, 1, _D), lambda i: (i, 0, 0)),
            pl.BlockSpec((_BB, 1, _D), lambda i: (i, 0, 0)),
            pl.BlockSpec((_BB, _D), lambda i: (i, 0)),
            pl.BlockSpec((_BB, _D), lambda i: (i, 0)),
        ],
        out_specs=[
            pl.BlockSpec((_BB, _MEM), lambda i: (i, 0)),
            pl.BlockSpec((_BB, _MEM), lambda i: (i, 0)),
            pl.BlockSpec((_BB, _MEM), lambda i: (i, 0)),
            pl.BlockSpec((_BB, _D), lambda i: (i, 0)),
        ],
        out_shape=[
            jax.ShapeDtypeStruct((_BS, _MEM), jnp.float32),
            jax.ShapeDtypeStruct((_BS, _MEM), jnp.float32),
            jax.ShapeDtypeStruct((_BS, _MEM), jnp.float32),
            jax.ShapeDtypeStruct((_BS, _D), jnp.float32),
        ],
    )(e, e, e, e, dh_outputs, kb2, conv2, u0, q)


@jax.jit
def kernel(story, kb_len, conv_len, dh_hidden, dh_outputs, query_vector, C):
    table = C.reshape(_NT * _VOCAB, _D)
    kb2 = jnp.broadcast_to(kb_len[:, None, None], (_B, 1, _D)).astype(jnp.int32)
    conv2 = jnp.broadcast_to(conv_len[:, None, None],
                             (_B, 1, _D)).astype(jnp.int32)
    u0 = dh_hidden[0]
    toff = (jnp.arange(_NT, dtype=jnp.int32) * _VOCAB)[:, None]

    outs = []
    for sl in range(_NSLICE):
        b0 = sl * _BS
        flat = story[b0:b0 + _BS].reshape(-1).astype(jnp.int32)
        idx = toff + flat[None, :]
        idx = idx.reshape(_NW, _N_CHUNKS, _CH_IN)
        e = _sc_gather_pool()(idx, table)
        outs.append(_tc_attention(
            e, dh_outputs[b0:b0 + _BS], kb2[b0:b0 + _BS], conv2[b0:b0 + _BS],
            u0[b0:b0 + _BS], query_vector[b0:b0 + _BS]))

    prob_soft, prob_logits, global_pointer, kb_readout = (
        jnp.concatenate([o[k] for o in outs], axis=0) for k in range(4))
    return prob_soft, prob_logits, global_pointer, kb_readout


# speedup vs baseline: 1.1178x; 1.0005x over previous
import functools

import jax
import jax.numpy as jnp
from jax import lax
from jax.experimental import pallas as pl
from jax.experimental.pallas import tpu as pltpu
from jax.experimental.pallas import tpu_sc as plsc

_VOCAB = 100000
_D = 128
_MAX_HOP = 3
_B = 128
_MEM = 200
_TOK = 4
_CONV_MAX = 100

_NT = _MAX_HOP + 1

_NC = 2
_NS = 16
_NW = _NC * _NS

_NSLICE = 4
_BS = _B // _NSLICE
_N_OUT = _NT * _BS * _MEM
_N_IN = _N_OUT * _TOK

_CH_IN = 128
_CH_OUT = _CH_IN // _TOK
_OUT_PER_W = _N_OUT // _NW
_N_CHUNKS = _N_IN // _NW // _CH_IN
_N_PAIRS = _N_CHUNKS // 2
_HAS_TAIL = _N_CHUNKS % 2

_BB = 16


def _sc_body(idx_hbm, table_hbm, out_hbm,
             idx_v, rows_a, rows_b, out_a, out_b,
             sem_a, sem_b, sem_wa, sem_wb):
    c = lax.axis_index("c")
    s = lax.axis_index("s")
    wid = s * _NC + c
    out_base = wid * _OUT_PER_W

    pltpu.sync_copy(idx_hbm.at[wid], idx_v)

    def fire_gather(buf, sem, g):
        pltpu.async_copy(table_hbm.at[idx_v.at[g]], buf, sem)

    def drain_gather(buf, sem):
        pltpu.make_async_copy(table_hbm.at[pl.ds(0, _CH_IN)], buf, sem).wait()

    def drain_write(buf, sem):
        ob = pl.multiple_of(out_base, 8)
        pltpu.make_async_copy(buf, out_hbm.at[pl.ds(ob, _CH_OUT)],
                              sem).wait()

    def sum_chunk(buf, out):
        def row_body(r, carry):
            for cc in range(_D // 16):
                sl = pl.ds(cc * 16, 16)
                out[r, sl] = (buf[4 * r, sl] + buf[4 * r + 1, sl]
                              + buf[4 * r + 2, sl] + buf[4 * r + 3, sl])
            return carry
        lax.fori_loop(0, _CH_OUT, row_body, 0, unroll=8)

    fire_gather(rows_a, sem_a, 0)

    def pair_body(i, carry):
        ga = 2 * i
        gb = ga + 1
        fire_gather(rows_b, sem_b, gb)

        drain_gather(rows_a, sem_a)

        @pl.when(i > 0)
        def _():
            drain_write(out_a, sem_wa)
        sum_chunk(rows_a, out_a)
        pltpu.async_copy(
            out_a,
            out_hbm.at[pl.ds(pl.multiple_of(out_base + ga * _CH_OUT, 8),
                             _CH_OUT)],
            sem_wa)

        @pl.when(ga + 2 < _N_CHUNKS)
        def _():
            fire_gather(rows_a, sem_a, ga + 2)

        drain_gather(rows_b, sem_b)

        @pl.when(i > 0)
        def _():
            drain_write(out_b, sem_wb)
        sum_chunk(rows_b, out_b)
        pltpu.async_copy(
            out_b,
            out_hbm.at[pl.ds(pl.multiple_of(out_base + gb * _CH_OUT, 8),
                             _CH_OUT)],
            sem_wb)
        return carry

    lax.fori_loop(0, _N_PAIRS, pair_body, 0, unroll=False)
    if _HAS_TAIL:
        gt = _N_CHUNKS - 1
        drain_gather(rows_a, sem_a)
        drain_write(out_a, sem_wa)
        sum_chunk(rows_a, out_a)
        pltpu.async_copy(
            out_a,
            out_hbm.at[pl.ds(pl.multiple_of(out_base + gt * _CH_OUT, 8),
                             _CH_OUT)],
            sem_wa)
    drain_write(out_a, sem_wa)
    drain_write(out_b, sem_wb)


@functools.lru_cache(maxsize=1)
def _sc_gather_pool():
    return pl.kernel(
        _sc_body,
        mesh=plsc.VectorSubcoreMesh(core_axis_name="c", subcore_axis_name="s",
                                    num_cores=_NC, num_subcores=_NS),
        out_type=jax.ShapeDtypeStruct((_N_OUT, _D), jnp.float32),
        scratch_types=[
            pltpu.VMEM((_N_CHUNKS, _CH_IN), jnp.int32),
            pltpu.VMEM((_CH_IN, _D), jnp.float32),
            pltpu.VMEM((_CH_IN, _D), jnp.float32),
            pltpu.VMEM((_CH_OUT, _D), jnp.float32),
            pltpu.VMEM((_CH_OUT, _D), jnp.float32),
            pltpu.SemaphoreType.DMA,
            pltpu.SemaphoreType.DMA,
            pltpu.SemaphoreType.DMA,
            pltpu.SemaphoreType.DMA,
        ],
    )


def _tc_body(e0_ref, e1_ref, e2_ref, e3_ref, dh_ref, kb_ref, conv_ref,
             u0_ref, q_ref, ps_ref, plg_ref, gp_ref, kr_ref):
    dh = dh_ref[...]
    kb3 = kb_ref[...][:, :, :1]
    conv3 = conv_ref[...][:, :, :1]

    m_iota3 = lax.broadcasted_iota(jnp.int32, (_BB, _MEM, _CONV_MAX), 1)
    j_iota3 = lax.broadcasted_iota(jnp.int32, (_BB, _MEM, _CONV_MAX), 2)
    hid3 = m_iota3 - kb3
    onehot = ((j_iota3 == hid3) & (hid3 >= 0)
              & (hid3 < conv3)).astype(jnp.float32)
    lm = lax.dot_general(onehot, dh, (((2,), (1,)), ((0,), (0,))),
                         precision=lax.Precision.HIGHEST)

    lm_flat = lm.reshape(_BB * _MEM, _D)
    Eb = [e0_ref[...] + lm_flat, e1_ref[...] + lm_flat,
          e2_ref[...] + lm_flat, e3_ref[...] + lm_flat]
    Eb = [e.reshape(_BB, _MEM, _D) for e in Eb]

    def attend(e, u):
        return lax.dot_general(e, u, (((2,), (1,)), ((0,), (0,))),
                               precision=lax.Precision.HIGHEST)

    def pool(e, p):
        return lax.dot_general(p, e, (((1,), (1,)), ((0,), (0,))),
                               precision=lax.Precision.HIGHEST)

    u = u0_ref[...]
    logit = None
    for hop in range(_MAX_HOP):
        logit = attend(Eb[hop], u)
        p = jax.nn.softmax(logit, axis=1)
        u = u + pool(Eb[hop + 1], p)
    gp = jax.nn.sigmoid(logit)
    gp_ref[...] = gp
    kr_ref[...] = u

    u2 = q_ref[...]
    p2 = None
    logits2 = None
    for hop in range(_MAX_HOP):
        logits2 = gp * attend(Eb[hop], u2)
        p2 = jax.nn.softmax(logits2, axis=1)
        u2 = u2 + pool(Eb[hop + 1], p2 * gp)
    ps_ref[...] = p2
    plg_ref[...] = logits2


def _e_spec(t):
    return pl.BlockSpec((_BB * _MEM, _D),
                        lambda i, t=t: (t * (_BS // _BB) + i, 0))


def _tc_attention(e, dh_outputs, kb2, conv2, u0, q):
    grid = (_BS // _BB,)
    return pl.pallas_call(
        _tc_body,
        grid=grid,
        in_specs=[
            _e_spec(0), _e_spec(1), _e_spec(2), _e_spec(3),
            pl.BlockSpec((_BB, _CONV_MAX, _D), lambda i: (i, 0, 0)),
            pl.BlockSpec((_BB, 1, _D), lambda i: (i, 0, 0)),
            pl.BlockSpec((_BB, 1, _D), lambda i: (i, 0, 0)),
            pl.BlockSpec((_BB, _D), lambda i: (i, 0)),
            pl.BlockSpec((_BB, _D), lambda i: (i, 0)),
        ],
        out_specs=[
            pl.BlockSpec((_BB, _MEM), lambda i: (i, 0)),
            pl.BlockSpec((_BB, _MEM), lambda i: (i, 0)),
            pl.BlockSpec((_BB, _MEM), lambda i: (i, 0)),
            pl.BlockSpec((_BB, _D), lambda i: (i, 0)),
        ],
        out_shape=[
            jax.ShapeDtypeStruct((_BS, _MEM), jnp.float32),
            jax.ShapeDtypeStruct((_BS, _MEM), jnp.float32),
            jax.ShapeDtypeStruct((_BS, _MEM), jnp.float32),
            jax.ShapeDtypeStruct((_BS, _D), jnp.float32),
        ],
    )(e, e, e, e, dh_outputs, kb2, conv2, u0, q)


@jax.jit
def kernel(story, kb_len, conv_len, dh_hidden, dh_outputs, query_vector, C):
    table = C.reshape(_NT * _VOCAB, _D)
    kb2 = jnp.broadcast_to(kb_len[:, None, None], (_B, 1, _D)).astype(jnp.int32)
    conv2 = jnp.broadcast_to(conv_len[:, None, None],
                             (_B, 1, _D)).astype(jnp.int32)
    u0 = dh_hidden[0]
    toff = (jnp.arange(_NT, dtype=jnp.int32) * _VOCAB)[:, None]

    outs = []
    for sl in range(_NSLICE):
        b0 = sl * _BS
        flat = story[b0:b0 + _BS].reshape(-1).astype(jnp.int32)
        idx = toff + flat[None, :]
        idx = idx.reshape(_NW, _N_CHUNKS, _CH_IN)
        e = _sc_gather_pool()(idx, table)
        outs.append(_tc_attention(
            e, dh_outputs[b0:b0 + _BS], kb2[b0:b0 + _BS], conv2[b0:b0 + _BS],
            u0[b0:b0 + _BS], query_vector[b0:b0 + _BS]))

    prob_soft, prob_logits, global_pointer, kb_readout = (
        jnp.concatenate([o[k] for o in outs], axis=0) for k in range(4))
    return prob_soft, prob_logits, global_pointer, kb_readout


# speedup vs baseline: 1.1451x; 1.0244x over previous
import functools

import jax
import jax.numpy as jnp
from jax import lax
from jax.experimental import pallas as pl
from jax.experimental.pallas import tpu as pltpu
from jax.experimental.pallas import tpu_sc as plsc

_VOCAB = 100000
_D = 128
_MAX_HOP = 3
_B = 128
_MEM = 200
_TOK = 4
_CONV_MAX = 100

_NT = _MAX_HOP + 1

_NC = 2
_NS = 16
_NW = _NC * _NS

_NSLICE = 4
_BS = _B // _NSLICE
_N_OUT = _NT * _BS * _MEM
_N_IN = _N_OUT * _TOK

_CH_IN = 128
_CH_OUT = _CH_IN // _TOK
_OUT_PER_W = _N_OUT // _NW
_N_CHUNKS = _N_IN // _NW // _CH_IN
_N_PAIRS = _N_CHUNKS // 2
_HAS_TAIL = _N_CHUNKS % 2

_BB = 16


def _sc_body(idx_hbm, table_hbm, out_hbm,
             idx_v, rows_a, rows_b, out_a, out_b,
             sem_a, sem_b, sem_wa, sem_wb):
    c = lax.axis_index("c")
    s = lax.axis_index("s")
    wid = s * _NC + c
    out_base = wid * _OUT_PER_W

    pltpu.sync_copy(idx_hbm.at[wid], idx_v)

    def fire_gather(buf, sem, g):
        pltpu.async_copy(table_hbm.at[idx_v.at[g]], buf, sem)

    def drain_gather(buf, sem):
        pltpu.make_async_copy(table_hbm.at[pl.ds(0, _CH_IN)], buf, sem).wait()

    def drain_write(buf, sem):
        ob = pl.multiple_of(out_base, 8)
        pltpu.make_async_copy(buf, out_hbm.at[pl.ds(ob, _CH_OUT)],
                              sem).wait()

    def sum_chunk(buf, out):
        def row_body(r, carry):
            for cc in range(_D // 16):
                sl = pl.ds(cc * 16, 16)
                out[r, sl] = (buf[4 * r, sl] + buf[4 * r + 1, sl]
                              + buf[4 * r + 2, sl] + buf[4 * r + 3, sl])
            return carry
        lax.fori_loop(0, _CH_OUT, row_body, 0, unroll=4)

    fire_gather(rows_a, sem_a, 0)

    def pair_body(i, carry):
        ga = 2 * i
        gb = ga + 1
        fire_gather(rows_b, sem_b, gb)

        drain_gather(rows_a, sem_a)

        @pl.when(i > 0)
        def _():
            drain_write(out_a, sem_wa)
        sum_chunk(rows_a, out_a)
        pltpu.async_copy(
            out_a,
            out_hbm.at[pl.ds(pl.multiple_of(out_base + ga * _CH_OUT, 8),
                             _CH_OUT)],
            sem_wa)

        @pl.when(ga + 2 < _N_CHUNKS)
        def _():
            fire_gather(rows_a, sem_a, ga + 2)

        drain_gather(rows_b, sem_b)

        @pl.when(i > 0)
        def _():
            drain_write(out_b, sem_wb)
        sum_chunk(rows_b, out_b)
        pltpu.async_copy(
            out_b,
            out_hbm.at[pl.ds(pl.multiple_of(out_base + gb * _CH_OUT, 8),
                             _CH_OUT)],
            sem_wb)
        return carry

    lax.fori_loop(0, _N_PAIRS, pair_body, 0, unroll=False)
    if _HAS_TAIL:
        gt = _N_CHUNKS - 1
        drain_gather(rows_a, sem_a)
        drain_write(out_a, sem_wa)
        sum_chunk(rows_a, out_a)
        pltpu.async_copy(
            out_a,
            out_hbm.at[pl.ds(pl.multiple_of(out_base + gt * _CH_OUT, 8),
                             _CH_OUT)],
            sem_wa)
    drain_write(out_a, sem_wa)
    drain_write(out_b, sem_wb)


@functools.lru_cache(maxsize=1)
def _sc_gather_pool():
    return pl.kernel(
        _sc_body,
        mesh=plsc.VectorSubcoreMesh(core_axis_name="c", subcore_axis_name="s",
                                    num_cores=_NC, num_subcores=_NS),
        out_type=jax.ShapeDtypeStruct((_N_OUT, _D), jnp.float32),
        scratch_types=[
            pltpu.VMEM((_N_CHUNKS, _CH_IN), jnp.int32),
            pltpu.VMEM((_CH_IN, _D), jnp.float32),
            pltpu.VMEM((_CH_IN, _D), jnp.float32),
            pltpu.VMEM((_CH_OUT, _D), jnp.float32),
            pltpu.VMEM((_CH_OUT, _D), jnp.float32),
            pltpu.SemaphoreType.DMA,
            pltpu.SemaphoreType.DMA,
            pltpu.SemaphoreType.DMA,
            pltpu.SemaphoreType.DMA,
        ],
    )


def _tc_body(e0_ref, e1_ref, e2_ref, e3_ref, dh_ref, kb_ref, conv_ref,
             u0_ref, q_ref, ps_ref, plg_ref, gp_ref, kr_ref):
    dh = dh_ref[...]
    kb3 = kb_ref[...][:, :, :1]
    conv3 = conv_ref[...][:, :, :1]

    m_iota3 = lax.broadcasted_iota(jnp.int32, (_BB, _MEM, _CONV_MAX), 1)
    j_iota3 = lax.broadcasted_iota(jnp.int32, (_BB, _MEM, _CONV_MAX), 2)
    hid3 = m_iota3 - kb3
    onehot = ((j_iota3 == hid3) & (hid3 >= 0)
              & (hid3 < conv3)).astype(jnp.float32)
    lm = lax.dot_general(onehot, dh, (((2,), (1,)), ((0,), (0,))),
                         precision=lax.Precision.HIGHEST)

    lm_flat = lm.reshape(_BB * _MEM, _D)
    Eb = [e0_ref[...] + lm_flat, e1_ref[...] + lm_flat,
          e2_ref[...] + lm_flat, e3_ref[...] + lm_flat]
    Eb = [e.reshape(_BB, _MEM, _D) for e in Eb]

    def attend(e, u):
        return lax.dot_general(e, u, (((2,), (1,)), ((0,), (0,))),
                               precision=lax.Precision.HIGHEST)

    def pool(e, p):
        return lax.dot_general(p, e, (((1,), (1,)), ((0,), (0,))),
                               precision=lax.Precision.HIGHEST)

    u = u0_ref[...]
    logit = None
    for hop in range(_MAX_HOP):
        logit = attend(Eb[hop], u)
        p = jax.nn.softmax(logit, axis=1)
        u = u + pool(Eb[hop + 1], p)
    gp = jax.nn.sigmoid(logit)
    gp_ref[...] = gp
    kr_ref[...] = u

    u2 = q_ref[...]
    p2 = None
    logits2 = None
    for hop in range(_MAX_HOP):
        logits2 = gp * attend(Eb[hop], u2)
        p2 = jax.nn.softmax(logits2, axis=1)
        u2 = u2 + pool(Eb[hop + 1], p2 * gp)
    ps_ref[...] = p2
    plg_ref[...] = logits2


def _e_spec(t):
    return pl.BlockSpec((_BB * _MEM, _D),
                        lambda i, t=t: (t * (_BS // _BB) + i, 0))


def _tc_attention(e, dh_outputs, kb2, conv2, u0, q):
    grid = (_BS // _BB,)
    return pl.pallas_call(
        _tc_body,
        grid=grid,
        in_specs=[
            _e_spec(0), _e_spec(1), _e_spec(2), _e_spec(3),
            pl.BlockSpec((_BB, _CONV_MAX, _D), lambda i: (i, 0, 0)),
            pl.BlockSpec((_BB, 1, _D), lambda i: (i, 0, 0)),
            pl.BlockSpec((_BB, 1, _D), lambda i: (i, 0, 0)),
            pl.BlockSpec((_BB, _D), lambda i: (i, 0)),
            pl.BlockSpec((_BB, _D), lambda i: (i, 0)),
        ],
        out_specs=[
            pl.BlockSpec((_BB, _MEM), lambda i: (i, 0)),
            pl.BlockSpec((_BB, _MEM), lambda i: (i, 0)),
            pl.BlockSpec((_BB, _MEM), lambda i: (i, 0)),
            pl.BlockSpec((_BB, _D), lambda i: (i, 0)),
        ],
        out_shape=[
            jax.ShapeDtypeStruct((_BS, _MEM), jnp.float32),
            jax.ShapeDtypeStruct((_BS, _MEM), jnp.float32),
            jax.ShapeDtypeStruct((_BS, _MEM), jnp.float32),
            jax.ShapeDtypeStruct((_BS, _D), jnp.float32),
        ],
    )(e, e, e, e, dh_outputs, kb2, conv2, u0, q)


@jax.jit
def kernel(story, kb_len, conv_len, dh_hidden, dh_outputs, query_vector, C):
    table = C.reshape(_NT * _VOCAB, _D)
    kb2 = jnp.broadcast_to(kb_len[:, None, None], (_B, 1, _D)).astype(jnp.int32)
    conv2 = jnp.broadcast_to(conv_len[:, None, None],
                             (_B, 1, _D)).astype(jnp.int32)
    u0 = dh_hidden[0]
    toff = (jnp.arange(_NT, dtype=jnp.int32) * _VOCAB)[:, None]

    outs = []
    for sl in range(_NSLICE):
        b0 = sl * _BS
        flat = story[b0:b0 + _BS].reshape(-1).astype(jnp.int32)
        idx = toff + flat[None, :]
        idx = idx.reshape(_NW, _N_CHUNKS, _CH_IN)
        e = _sc_gather_pool()(idx, table)
        outs.append(_tc_attention(
            e, dh_outputs[b0:b0 + _BS], kb2[b0:b0 + _BS], conv2[b0:b0 + _BS],
            u0[b0:b0 + _BS], query_vector[b0:b0 + _BS]))

    prob_soft, prob_logits, global_pointer, kb_readout = (
        jnp.concatenate([o[k] for o in outs], axis=0) for k in range(4))
    return prob_soft, prob_logits, global_pointer, kb_readout
